# Initial kernel scaffold; baseline (speedup 1.0000x reference)
import functools

import jax
import jax.numpy as jnp
from jax import lax
from jax.experimental import pallas as pl
from jax.experimental.pallas import tpu as pltpu
from jax.experimental.pallas import tpu_sc as plsc

N = 10000
N2 = 20000
H = 128
HH = 64
E = 320000
NT = 16
NCH = 158
EPT = NCH * 128
EPAD = NT * EPT
DT = 12.0 / 2.0
RB = 2000
GRID = N2 // RB
DEGB = 10240

_f32 = jnp.float32


def _deg_call(col3, w3):
    mesh = plsc.VectorSubcoreMesh(core_axis_name="c", subcore_axis_name="s")

    @functools.partial(
        pl.kernel,
        mesh=mesh,
        out_type=jax.ShapeDtypeStruct((DEGB,), _f32),
        scratch_types=[
            pltpu.VMEM((NCH, 128), jnp.int32),
            pltpu.VMEM((NCH, 128), _f32),
            pltpu.VMEM((DEGB,), _f32),
            pltpu.VMEM((NT, DEGB // NT), _f32),
            pltpu.VMEM((DEGB // NT,), _f32),
            pltpu.VMEM_SHARED((NT, DEGB), _f32),
        ],
    )
    def degk(col_hbm, w_hbm, out_hbm, col_v, w_v, hist, red, outb, shard):
        c = lax.axis_index("c")
        s = lax.axis_index("s")

        @pl.when(c == 0)
        def _():
            pltpu.sync_copy(col_hbm.at[s], col_v)
            pltpu.sync_copy(w_hbm.at[s], w_v)

            zero = jnp.zeros((16,), _f32)

            def zbody(i, carry):
                hist[pl.ds(i * 16, 16)] = zero
                return carry

            lax.fori_loop(0, DEGB // 16, zbody, 0)

            def hbody(j, carry):
                for k in range(8):
                    cv = col_v[j, pl.ds(k * 16, 16)]
                    wv = w_v[j, pl.ds(k * 16, 16)]
                    plsc.addupdate_scatter(hist, [cv], wv)
                return carry

            lax.fori_loop(0, NCH, hbody, 0)

            pltpu.sync_copy(hist, shard.at[s])
            plsc.subcore_barrier()

            base = s * (DEGB // NT)
            for k in range(NT):
                pltpu.sync_copy(shard.at[k, pl.ds(base, DEGB // NT)],
                                red.at[k])
            one = jnp.full((16,), 1.0, _f32)
            for i in range(DEGB // NT // 16):
                v = red[0, pl.ds(i * 16, 16)]
                for k in range(1, NT):
                    v = v + red[k, pl.ds(i * 16, 16)]
                outb[pl.ds(i * 16, 16)] = v + one
            pltpu.sync_copy(outb, out_hbm.at[pl.ds(base, DEGB // NT)])

    return degk(col3, w3)


def _spmm_call(T2, row3, col3, w3):
    mesh = plsc.VectorSubcoreMesh(core_axis_name="c", subcore_axis_name="s")

    @functools.partial(
        pl.kernel,
        mesh=mesh,
        out_type=jax.ShapeDtypeStruct((2, N2, HH), _f32),
        scratch_types=[
            pltpu.VMEM((NCH, 128), jnp.int32),
            pltpu.VMEM((NCH, 128), jnp.int32),
            pltpu.VMEM((NCH, 128), _f32),
            pltpu.VMEM((128, HH), _f32),
            pltpu.VMEM_SHARED((N2, HH), _f32),
            pltpu.SemaphoreType.DMA,
        ],
    )
    def spmm(t_hbm, row_hbm, col_hbm, w_hbm, out_hbm,
             row_v, col_v, w_v, rows, acc, gsem):
        c = lax.axis_index("c")
        s = lax.axis_index("s")
        tbl = t_hbm.at[c]
        nrows = N2 // NT
        sl = pl.ds(s * nrows, nrows)

        pltpu.sync_copy(row_hbm.at[s], row_v)
        pltpu.sync_copy(col_hbm.at[s], col_v)
        pltpu.sync_copy(w_hbm.at[s], w_v)
        pltpu.sync_copy(tbl.at[sl], acc.at[sl])
        plsc.subcore_barrier()

        def chunk(j, carry):
            pltpu.async_copy(tbl.at[row_v.at[j]], rows, gsem).wait()
            jj = jnp.full((16,), 0, jnp.int32) + j
            for e in range(128):
                wsp = plsc.load_gather(
                    w_v, [jj, jnp.full((16,), e, jnp.int32)])
                for fg in range(HH // 16):
                    rows[e, pl.ds(fg * 16, 16)] = (
                        rows[e, pl.ds(fg * 16, 16)] * wsp)
            pltpu.sync_copy(rows, acc.at[col_v.at[j]], add=True)
            return carry

        lax.fori_loop(0, NCH, chunk, 0)
        plsc.subcore_barrier()
        pltpu.sync_copy(acc.at[sl], out_hbm.at[c].at[sl])

    return spmm(T2, row3, col3, w3)


def _w_spec(shape):
    return pl.BlockSpec(shape, lambda i: (0,) * len(shape))


_ROW = pl.BlockSpec((RB, H), lambda i: (i, 0))
_ROW1 = pl.BlockSpec((RB, 1), lambda i: (i, 0))
_HALF = pl.BlockSpec((2, RB, HH), lambda i: (0, i, 0))


def _dot(a, b):
    return jnp.dot(a, b, preferred_element_type=_f32)


def _split(t, out_ref):
    out_ref[0] = t[:, :HH]
    out_ref[1] = t[:, HH:]


def _tc_prep(x, deg, w1, b1, w2, b2, gw):
    def body(x_ref, deg_ref, w1_ref, b1_ref, w2_ref, b2_ref, gw_ref, out_ref):
        h = jnp.maximum(x_ref[...] * w1_ref[...] + b1_ref[...], 0.0)
        h = _dot(h, w2_ref[...]) + b2_ref[...]
        t = lax.rsqrt(deg_ref[...]) * _dot(h, gw_ref[...])
        _split(t, out_ref)

    return pl.pallas_call(
        body,
        grid=(GRID,),
        in_specs=[_ROW1, _ROW1, _w_spec((1, H)), _w_spec((1, H)),
                  _w_spec((H, H)), _w_spec((1, H)), _w_spec((H, H))],
        out_specs=_HALF,
        out_shape=jax.ShapeDtypeStruct((2, N2, HH), _f32),
    )(x, deg, w1, b1, w2, b2, gw)


def _tc_h0(s, deg, gb):
    def body(s_ref, deg_ref, gb_ref, out_ref):
        sf = jnp.concatenate([s_ref[0], s_ref[1]], axis=1)
        out_ref[...] = jnp.maximum(
            lax.rsqrt(deg_ref[...]) * sf + gb_ref[...], 0.0)

    return pl.pallas_call(
        body,
        grid=(GRID,),
        in_specs=[_HALF, _ROW1, _w_spec((1, H))],
        out_specs=_ROW,
        out_shape=jax.ShapeDtypeStruct((N2, H), _f32),
    )(s, deg, gb)


def _tc_a(x, deg, mw1, mb1, mw2, mb2, gw1):
    def body(x_ref, deg_ref, mw1_ref, mb1_ref, mw2_ref, mb2_ref, gw1_ref,
             hn_ref, t_ref):
        x = x_ref[...]
        hn = _dot(jnp.tanh(_dot(x, mw1_ref[...]) + mb1_ref[...]),
                  mw2_ref[...]) + mb2_ref[...]
        hn_ref[...] = hn
        t = lax.rsqrt(deg_ref[...]) * _dot(x, gw1_ref[...])
        _split(t, t_ref)

    return pl.pallas_call(
        body,
        grid=(GRID,),
        in_specs=[_ROW, _ROW1, _w_spec((H, H)), _w_spec((1, H)),
                  _w_spec((H, H)), _w_spec((1, H)), _w_spec((H, H))],
        out_specs=[_ROW, _HALF],
        out_shape=[jax.ShapeDtypeStruct((N2, H), _f32),
                   jax.ShapeDtypeStruct((2, N2, HH), _f32)],
    )(x, deg, mw1, mb1, mw2, mb2, gw1)


def _tc_b(s, deg, g1b, gw2):
    def body(s_ref, deg_ref, g1b_ref, gw2_ref, t_ref):
        dinv = lax.rsqrt(deg_ref[...])
        sf = jnp.concatenate([s_ref[0], s_ref[1]], axis=1)
        hg1 = jnp.maximum(dinv * sf + g1b_ref[...], 0.0)
        t = dinv * _dot(hg1, gw2_ref[...])
        _split(t, t_ref)

    return pl.pallas_call(
        body,
        grid=(GRID,),
        in_specs=[_HALF, _ROW1, _w_spec((1, H)), _w_spec((H, H))],
        out_specs=_HALF,
        out_shape=jax.ShapeDtypeStruct((2, N2, HH), _f32),
    )(s, deg, g1b, gw2)


def _tc_c(s, deg, g2b, hn, gwa, gwb, gb, hcur, ksum, a_next, wk, last):
    def body(s_ref, deg_ref, g2b_ref, hn_ref, gwa_ref, gwb_ref, gb_ref,
             hcur_ref, *rest):
        if ksum is not None:
            ksum_ref = rest[0]
            rest = rest[1:]
        if last:
            (hnew_ref,) = rest
        else:
            xn_ref, ks_ref = rest
        sf = jnp.concatenate([s_ref[0], s_ref[1]], axis=1)
        hg = lax.rsqrt(deg_ref[...]) * sf + g2b_ref[...]
        hn = hn_ref[...]
        g = jax.nn.sigmoid(_dot(hg, gwa_ref[...]) + _dot(hn, gwb_ref[...])
                           + gb_ref[...])
        k = g * hg + (1.0 - g) * hn
        if last:
            hnew_ref[...] = hcur_ref[...] + (DT / 6.0) * (ksum_ref[...] + k)
        else:
            xn_ref[...] = hcur_ref[...] + a_next * k
            if ksum is None:
                ks_ref[...] = wk * k
            else:
                ks_ref[...] = ksum_ref[...] + wk * k

    in_specs = [_HALF, _ROW1, _w_spec((1, H)), _ROW, _w_spec((H, H)),
                _w_spec((H, H)), _w_spec((1, H)), _ROW]
    args = [s, deg, g2b, hn, gwa, gwb, gb, hcur]
    if ksum is not None:
        in_specs.append(_ROW)
        args.append(ksum)
    if last:
        out_specs = _ROW
        out_shape = jax.ShapeDtypeStruct((N2, H), _f32)
    else:
        out_specs = [_ROW, _ROW]
        out_shape = [jax.ShapeDtypeStruct((N2, H), _f32),
                     jax.ShapeDtypeStruct((N2, H), _f32)]
    return pl.pallas_call(
        body,
        grid=(GRID,),
        in_specs=in_specs,
        out_specs=out_specs,
        out_shape=out_shape,
    )(*args)


def _tc_dec(hcur, w1, b1, w2, b2):
    def body(h_ref, w1_ref, b1_ref, w2_ref, b2_ref, out_ref):
        d = jnp.maximum(_dot(h_ref[...], w1_ref[...]) + b1_ref[...], 0.0)
        out_ref[...] = _dot(d, w2_ref[...]) + b2_ref[...]

    return pl.pallas_call(
        body,
        grid=(GRID,),
        in_specs=[_ROW, _w_spec((H, H)), _w_spec((1, H)),
                  _w_spec((H, 1)), _w_spec((1, 1))],
        out_specs=_ROW1,
        out_shape=jax.ShapeDtypeStruct((N2, 1), _f32),
    )(hcur, w1, b1, w2, b2)


def kernel(X, edge_index, edge_weight, enc_W1, enc_b1, enc_W2, enc_b2,
           gcn_W, gcn_b, mlp_W1, mlp_b1, mlp_W2, mlp_b2,
           gc1_W, gc1_b, gc2_W, gc2_b, gate_W, gate_b,
           dec_W1, dec_b1, dec_W2, dec_b2):
    row = edge_index[0]
    col = edge_index[1]
    pad = EPAD - E
    row3 = jnp.pad(row, (0, pad)).reshape(NT, NCH, 128)
    col3 = jnp.pad(col, (0, pad)).reshape(NT, NCH, 128)
    w3 = jnp.pad(edge_weight, (0, pad)).reshape(NT, NCH, 128)

    x2d = X[:, -1, :, :].reshape(N2, 1)

    def r2(b):
        return b.reshape(1, -1)

    gwa = gate_W[:H]
    gwb = gate_W[H:]

    degh = _deg_call(col3, w3)
    deg = jnp.concatenate([degh[:N], jnp.ones((N,), _f32)]).reshape(N2, 1)

    t0 = _tc_prep(x2d, deg, r2(enc_W1[0]), r2(enc_b1), enc_W2, r2(enc_b2),
                  gcn_W)
    s0 = _spmm_call(t0, row3, col3, w3)
    hcur = _tc_h0(s0, deg, r2(gcn_b))

    for _ in range(2):
        ksum = None
        x = hcur
        for i in range(4):
            hn, t1 = _tc_a(x, deg, mlp_W1, r2(mlp_b1), mlp_W2, r2(mlp_b2),
                           gc1_W)
            s1 = _spmm_call(t1, row3, col3, w3)
            t2 = _tc_b(s1, deg, r2(gc1_b), gc2_W)
            s2 = _spmm_call(t2, row3, col3, w3)
            if i < 3:
                a_next = 0.5 * DT if i < 2 else DT
                wk = 1.0 if i == 0 else 2.0
                x, ksum = _tc_c(s2, deg, r2(gc2_b), hn, gwa, gwb, r2(gate_b),
                                hcur, ksum, a_next, wk, last=False)
            else:
                hcur = _tc_c(s2, deg, r2(gc2_b), hn, gwa, gwb, r2(gate_b),
                             hcur, ksum, 0.0, 1.0, last=True)

    out = _tc_dec(hcur, dec_W1, r2(dec_b1), dec_W2, r2(dec_b2))
    return out.reshape(2, N, 1)[:, None, :, :]


# speedup vs baseline: 4.8191x; 4.8191x over previous
import functools

import jax
import jax.numpy as jnp
from jax import lax
from jax.experimental import pallas as pl
from jax.experimental.pallas import tpu as pltpu
from jax.experimental.pallas import tpu_sc as plsc

N = 10000
N2 = 20000
N2P = 20480
NLOW = 10240
H = 128
E = 320000
NT = 16
NW = 32
NCH = 79
EPAD = NW * NCH * 128
DT = 12.0 / 2.0
RB = 2048
GRID = N2P // RB
LOWB = NLOW // RB

_f32 = jnp.float32

_GDN = lax.GatherDimensionNumbers(
    offset_dims=(), collapsed_slice_dims=(0,), start_index_map=(0,))


def _bcast_lane(vec, e):
    idx = jnp.full((16, 1), e, jnp.int32)
    return lax.gather(vec, idx, _GDN, slice_sizes=(1,),
                      mode=lax.GatherScatterMode.PROMISE_IN_BOUNDS)


def _spmm_call(T, row3, col3, w3, with_gather=True):
    mesh = plsc.VectorSubcoreMesh(core_axis_name="c", subcore_axis_name="s")

    @functools.partial(
        pl.kernel,
        mesh=mesh,
        out_type=jax.ShapeDtypeStruct((2, NLOW, H), _f32),
        scratch_types=[
            pltpu.VMEM((NCH, 128), jnp.int32),
            pltpu.VMEM((NCH, 128), jnp.int32),
            pltpu.VMEM((NCH, 128), _f32),
            pltpu.VMEM((128, H), _f32),
            pltpu.VMEM_SHARED((NLOW, H), _f32),
            pltpu.SemaphoreType.DMA,
        ],
    )
    def spmm(t_hbm, row_hbm, col_hbm, w_hbm, out_hbm,
             row_v, col_v, w_v, rows, acc, gsem):
        c = lax.axis_index("c")
        s = lax.axis_index("s")
        eslice = c * NT + s
        nrt = NLOW // NT
        sl = pl.ds(s * nrt, nrt)

        pltpu.sync_copy(row_hbm.at[eslice], row_v)
        pltpu.sync_copy(col_hbm.at[eslice], col_v)
        pltpu.sync_copy(w_hbm.at[eslice], w_v)

        @pl.when(c == 0)
        def _():
            pltpu.sync_copy(t_hbm.at[sl], acc.at[sl])

        @pl.when(c == 1)
        def _():
            z = jnp.zeros((16,), _f32)
            for r in range(128):
                rr = rows.at[r]
                for fg in range(H // 16):
                    rr[pl.ds(fg * 16, 16)] = z
            for q in range(nrt // 128):
                pltpu.sync_copy(rows, acc.at[pl.ds(s * nrt + q * 128, 128)])

        plsc.subcore_barrier()

        def chunk(j, carry):
            if with_gather:
                pltpu.async_copy(t_hbm.at[row_v.at[j]], rows, gsem).wait()
            wrow = w_v.at[j]
            for k in range(8):
                wvec = wrow[pl.ds(k * 16, 16)]
                for e in range(16):
                    wsp = _bcast_lane(wvec, e)
                    rr = rows.at[k * 16 + e]
                    for fg in range(H // 16):
                        if with_gather:
                            rr[pl.ds(fg * 16, 16)] = (
                                rr[pl.ds(fg * 16, 16)] * wsp)
                        else:
                            rr[pl.ds(fg * 16, 16)] = wsp
            pltpu.sync_copy(rows, acc.at[col_v.at[j]], add=True)
            return carry

        lax.fori_loop(0, NCH, chunk, 0)
        plsc.subcore_barrier()
        pltpu.sync_copy(acc.at[sl], out_hbm.at[c].at[sl])

    return spmm(T, row3, col3, w3)


def _w_spec(shape):
    return pl.BlockSpec(shape, lambda i: (0,) * len(shape))


_ROW = pl.BlockSpec((RB, H), lambda i: (i, 0))
_ROW1 = pl.BlockSpec((RB, 1), lambda i: (i, 0))
_PLOW = pl.BlockSpec((2, RB, H), lambda i: (0, jnp.minimum(i, LOWB - 1), 0))


def _dot(a, b):
    return jnp.dot(a, b, preferred_element_type=_f32)


def _combine(p_ref, t_ref):
    low = p_ref[0] + p_ref[1]
    return jnp.where(pl.program_id(0) < LOWB, low, t_ref[...])


def _tc_prep(x, deg, w1, b1, w2, b2, gw):
    def body(x_ref, deg_ref, w1_ref, b1_ref, w2_ref, b2_ref, gw_ref, out_ref):
        h = jnp.maximum(x_ref[...] * w1_ref[...] + b1_ref[...], 0.0)
        h = _dot(h, w2_ref[...]) + b2_ref[...]
        out_ref[...] = lax.rsqrt(deg_ref[...]) * _dot(h, gw_ref[...])

    return pl.pallas_call(
        body,
        grid=(GRID,),
        in_specs=[_ROW1, _ROW1, _w_spec((1, H)), _w_spec((1, H)),
                  _w_spec((H, H)), _w_spec((1, H)), _w_spec((H, H))],
        out_specs=_ROW,
        out_shape=jax.ShapeDtypeStruct((N2P, H), _f32),
    )(x, deg, w1, b1, w2, b2, gw)


def _tc_h0(p, t, deg, gb):
    def body(p_ref, t_ref, deg_ref, gb_ref, out_ref):
        sf = _combine(p_ref, t_ref)
        out_ref[...] = jnp.maximum(
            lax.rsqrt(deg_ref[...]) * sf + gb_ref[...], 0.0)

    return pl.pallas_call(
        body,
        grid=(GRID,),
        in_specs=[_PLOW, _ROW, _ROW1, _w_spec((1, H))],
        out_specs=_ROW,
        out_shape=jax.ShapeDtypeStruct((N2P, H), _f32),
    )(p, t, deg, gb)


def _tc_a(x, deg, mw1, mb1, mw2, mb2, gw1):
    def body(x_ref, deg_ref, mw1_ref, mb1_ref, mw2_ref, mb2_ref, gw1_ref,
             hn_ref, t_ref):
        x = x_ref[...]
        hn = _dot(jnp.tanh(_dot(x, mw1_ref[...]) + mb1_ref[...]),
                  mw2_ref[...]) + mb2_ref[...]
        hn_ref[...] = hn
        t_ref[...] = lax.rsqrt(deg_ref[...]) * _dot(x, gw1_ref[...])

    return pl.pallas_call(
        body,
        grid=(GRID,),
        in_specs=[_ROW, _ROW1, _w_spec((H, H)), _w_spec((1, H)),
                  _w_spec((H, H)), _w_spec((1, H)), _w_spec((H, H))],
        out_specs=[_ROW, _ROW],
        out_shape=[jax.ShapeDtypeStruct((N2P, H), _f32),
                   jax.ShapeDtypeStruct((N2P, H), _f32)],
    )(x, deg, mw1, mb1, mw2, mb2, gw1)


def _tc_b(p, t, deg, g1b, gw2):
    def body(p_ref, t_ref, deg_ref, g1b_ref, gw2_ref, out_ref):
        dinv = lax.rsqrt(deg_ref[...])
        sf = _combine(p_ref, t_ref)
        hg1 = jnp.maximum(dinv * sf + g1b_ref[...], 0.0)
        out_ref[...] = dinv * _dot(hg1, gw2_ref[...])

    return pl.pallas_call(
        body,
        grid=(GRID,),
        in_specs=[_PLOW, _ROW, _ROW1, _w_spec((1, H)), _w_spec((H, H))],
        out_specs=_ROW,
        out_shape=jax.ShapeDtypeStruct((N2P, H), _f32),
    )(p, t, deg, g1b, gw2)


def _tc_c(p, t, deg, g2b, hn, gwa, gwb, gb, hcur, ksum, a_next, wk, last):
    def body(p_ref, t_ref, deg_ref, g2b_ref, hn_ref, gwa_ref, gwb_ref,
             gb_ref, hcur_ref, *rest):
        if ksum is not None:
            ksum_ref = rest[0]
            rest = rest[1:]
        if last:
            (hnew_ref,) = rest
        else:
            xn_ref, ks_ref = rest
        hg = lax.rsqrt(deg_ref[...]) * _combine(p_ref, t_ref) + g2b_ref[...]
        hn = hn_ref[...]
        g = jax.nn.sigmoid(_dot(hg, gwa_ref[...]) + _dot(hn, gwb_ref[...])
                           + gb_ref[...])
        k = g * hg + (1.0 - g) * hn
        if last:
            hnew_ref[...] = hcur_ref[...] + (DT / 6.0) * (ksum_ref[...] + k)
        else:
            xn_ref[...] = hcur_ref[...] + a_next * k
            if ksum is None:
                ks_ref[...] = wk * k
            else:
                ks_ref[...] = ksum_ref[...] + wk * k

    in_specs = [_PLOW, _ROW, _ROW1, _w_spec((1, H)), _ROW, _w_spec((H, H)),
                _w_spec((H, H)), _w_spec((1, H)), _ROW]
    args = [p, t, deg, g2b, hn, gwa, gwb, gb, hcur]
    if ksum is not None:
        in_specs.append(_ROW)
        args.append(ksum)
    if last:
        out_specs = _ROW
        out_shape = jax.ShapeDtypeStruct((N2P, H), _f32)
    else:
        out_specs = [_ROW, _ROW]
        out_shape = [jax.ShapeDtypeStruct((N2P, H), _f32),
                     jax.ShapeDtypeStruct((N2P, H), _f32)]
    return pl.pallas_call(
        body,
        grid=(GRID,),
        in_specs=in_specs,
        out_specs=out_specs,
        out_shape=out_shape,
    )(*args)


def _tc_dec(hcur, w1, b1, w2, b2):
    def body(h_ref, w1_ref, b1_ref, w2_ref, b2_ref, out_ref):
        d = jnp.maximum(_dot(h_ref[...], w1_ref[...]) + b1_ref[...], 0.0)
        out_ref[...] = _dot(d, w2_ref[...]) + b2_ref[...]

    return pl.pallas_call(
        body,
        grid=(GRID,),
        in_specs=[_ROW, _w_spec((H, H)), _w_spec((1, H)),
                  _w_spec((H, 1)), _w_spec((1, 1))],
        out_specs=_ROW1,
        out_shape=jax.ShapeDtypeStruct((N2P, 1), _f32),
    )(hcur, w1, b1, w2, b2)


def kernel(X, edge_index, edge_weight, enc_W1, enc_b1, enc_W2, enc_b2,
           gcn_W, gcn_b, mlp_W1, mlp_b1, mlp_W2, mlp_b2,
           gc1_W, gc1_b, gc2_W, gc2_b, gate_W, gate_b,
           dec_W1, dec_b1, dec_W2, dec_b2):
    row = edge_index[0]
    col = edge_index[1]
    pad = EPAD - E
    row3 = jnp.pad(row, (0, pad)).reshape(NW, NCH, 128)
    col3 = jnp.pad(col, (0, pad)).reshape(NW, NCH, 128)
    w3 = jnp.pad(edge_weight, (0, pad)).reshape(NW, NCH, 128)

    x2d = jnp.pad(X[:, -1, :, :].reshape(N2, 1), ((0, N2P - N2), (0, 0)))

    def r2(b):
        return b.reshape(1, -1)

    gwa = gate_W[:H]
    gwb = gate_W[H:]

    onesT = jnp.ones((N2P, H), _f32)
    pdeg = _spmm_call(onesT, row3, col3, w3, with_gather=False)
    deg = jnp.concatenate(
        [pdeg[0, :, 0:1] + pdeg[1, :, 0:1],
         jnp.ones((N2P - NLOW, 1), _f32)])

    t0 = _tc_prep(x2d, deg, r2(enc_W1[0]), r2(enc_b1), enc_W2, r2(enc_b2),
                  gcn_W)
    p0 = _spmm_call(t0, row3, col3, w3)
    hcur = _tc_h0(p0, t0, deg, r2(gcn_b))

    for _ in range(2):
        ksum = None
        x = hcur
        for i in range(4):
            hn, t1 = _tc_a(x, deg, mlp_W1, r2(mlp_b1), mlp_W2, r2(mlp_b2),
                           gc1_W)
            p1 = _spmm_call(t1, row3, col3, w3)
            t2 = _tc_b(p1, t1, deg, r2(gc1_b), gc2_W)
            p2 = _spmm_call(t2, row3, col3, w3)
            if i < 3:
                a_next = 0.5 * DT if i < 2 else DT
                wk = 1.0 if i == 0 else 2.0
                x, ksum = _tc_c(p2, t2, deg, r2(gc2_b), hn, gwa, gwb,
                                r2(gate_b), hcur, ksum, a_next, wk,
                                last=False)
            else:
                hcur = _tc_c(p2, t2, deg, r2(gc2_b), hn, gwa, gwb,
                             r2(gate_b), hcur, ksum, 0.0, 1.0, last=True)

    out = _tc_dec(hcur, dec_W1, r2(dec_b1), dec_W2, r2(dec_b2))
    return out[:N2].reshape(2, N, 1)[:, None, :, :]
